# baseline (speedup 1.0000x reference)
import functools

import jax
import jax.numpy as jnp
from jax import lax
from jax.experimental import pallas as pl
from jax.experimental.pallas import tpu as pltpu
from jax.experimental.pallas import tpu_sc as plsc

_NC = 2
_NS = 16
_NW = _NC * _NS
_L = 16
_B = 16384
_F = 64
_BPW = _B // _NW
_G = _BPW // _L

_mesh = plsc.VectorSubcoreMesh(core_axis_name="c", subcore_axis_name="s")


@functools.partial(
    pl.kernel,
    out_type=jax.ShapeDtypeStruct((_B,), jnp.float32),
    mesh=_mesh,
    compiler_params=pltpu.CompilerParams(
        needs_layout_passes=False, use_tc_tiling_on_sc=False),
    scratch_types=[
        pltpu.VMEM((_BPW,), jnp.int32),
        pltpu.VMEM((_BPW,), jnp.int32),
        pltpu.VMEM((_BPW, _F), jnp.float32),
        pltpu.VMEM((_BPW, _F), jnp.float32),
        pltpu.VMEM((_BPW,), jnp.float32),
        pltpu.VMEM((_BPW,), jnp.float32),
        pltpu.VMEM((_BPW,), jnp.float32),
        pltpu.SemaphoreType.DMA,
        pltpu.SemaphoreType.DMA,
        pltpu.SemaphoreType.DMA,
        pltpu.SemaphoreType.DMA,
    ],
)
def _mf_score(users_hbm, items_hbm, ue_hbm, ie_hbm, ub_hbm, ib_hbm, out_hbm,
              uidx_v, iidx_v, ue_rows, ie_rows, ub_v, ib_v, out_v,
              sem0, sem1, sem2, sem3):
    wid = lax.axis_index("s") * _NC + lax.axis_index("c")
    base = wid * _BPW

    pltpu.sync_copy(users_hbm.at[pl.ds(base, _BPW)], uidx_v)
    pltpu.sync_copy(items_hbm.at[pl.ds(base, _BPW)], iidx_v)
    cp0 = pltpu.async_copy(ue_hbm.at[uidx_v], ue_rows, sem0)
    cp1 = pltpu.async_copy(ie_hbm.at[iidx_v], ie_rows, sem1)
    cp2 = pltpu.async_copy(ub_hbm.at[uidx_v], ub_v, sem2)
    cp3 = pltpu.async_copy(ib_hbm.at[iidx_v], ib_v, sem3)
    cp0.wait()
    cp1.wait()
    cp2.wait()
    cp3.wait()

    lanes = lax.iota(jnp.int32, _L)

    def group(g, carry):
        b0 = g * _L
        r = b0 + lanes
        accs = [jnp.zeros((_L,), jnp.float32) for _ in range(4)]
        for c in range(_F):
            cv = jnp.full((_L,), c, jnp.int32)
            accs[c % 4] = accs[c % 4] + (
                plsc.load_gather(ue_rows, [r, cv])
                * plsc.load_gather(ie_rows, [r, cv]))
        acc = ((accs[0] + accs[1]) + (accs[2] + accs[3])
               + ub_v[pl.ds(b0, _L)] + ib_v[pl.ds(b0, _L)])
        out_v[pl.ds(b0, _L)] = acc
        return carry

    lax.fori_loop(0, _G, group, 0)
    pltpu.sync_copy(out_v, out_hbm.at[pl.ds(base, _BPW)])


def kernel(users, items, user_embeddings, item_embeddings, user_biases,
           item_biases):
    ub = user_biases.reshape(-1)
    ib = item_biases.reshape(-1)
    out = _mf_score(users, items, user_embeddings, item_embeddings, ub, ib)
    return out.reshape(_B, 1)
